# baseline (speedup 1.0000x reference)
import functools

import jax
import jax.numpy as jnp
from jax import lax
from jax.experimental import pallas as pl
from jax.experimental.pallas import tpu as pltpu
from jax.experimental.pallas import tpu_sc as plsc

N = 10000
E = 320000
G = 200
D = 128
OUT = 4
PER = N // G
K = PER // 2
NC = 2
NS = 16
NW = NC * NS
EPW = E // NW
CH = 128
NCH = 79
PAD = NCH * CH - EPW
NP_ = 10240
NPT = NP_ // NS
NZ = NPT // CH

_mesh = plsc.VectorSubcoreMesh(core_axis_name="c", subcore_axis_name="s")


@functools.partial(
    pl.kernel,
    mesh=_mesh,
    out_type=[
        jax.ShapeDtypeStruct((NC, NP_, D), jnp.float32),
        jax.ShapeDtypeStruct((NC, NP_), jnp.float32),
    ],
    scratch_types=[
        pltpu.VMEM((CH,), jnp.int32),
        pltpu.VMEM((CH,), jnp.int32),
        pltpu.VMEM((NCH, CH), jnp.int32),
        pltpu.VMEM((CH, D), jnp.float32),
        pltpu.VMEM((CH, D), jnp.float32),
        pltpu.VMEM((CH,), jnp.float32),
        pltpu.VMEM((NPT,), jnp.float32),
        pltpu.VMEM_SHARED((NP_, D), jnp.float32),
        pltpu.VMEM_SHARED((NP_,), jnp.float32),
        pltpu.SemaphoreType.DMA,
        pltpu.SemaphoreType.DMA,
        pltpu.SemaphoreType.DMA,
        pltpu.SemaphoreType.DMA,
        pltpu.SemaphoreType.DMA,
        pltpu.SemaphoreType.DMA,
    ],
)
def _sc_pass1(x_hbm, src_hbm, dst_hbm, z128_hbm, zdeg_hbm, ones_hbm,
              agg_out, deg_out, srca, srcb2, dstb, rowsa, rowsb, onesb, stgd,
              agg_sh, deg_sh, sga, sgb, sia, sib, ssa, ssb):
    c = lax.axis_index("c")
    s = lax.axis_index("s")
    wid = c * NS + s
    pltpu.sync_copy(z128_hbm, rowsa)
    pltpu.sync_copy(zdeg_hbm, stgd)
    pltpu.sync_copy(ones_hbm, onesb)
    pltpu.sync_copy(dst_hbm.at[wid], dstb)
    for t in range(NZ):
        pltpu.sync_copy(rowsa, agg_sh.at[pl.ds(s * NPT + t * CH, CH)])
    pltpu.sync_copy(stgd, deg_sh.at[pl.ds(s * NPT, NPT)])
    pltpu.sync_copy(src_hbm.at[wid, 0], srca)
    pltpu.async_copy(x_hbm.at[srca], rowsa, sga)
    pltpu.async_copy(src_hbm.at[wid, 1], srcb2, sib)
    plsc.subcore_barrier()

    def body(t, carry):
        a = 2 * t
        b = a + 1

        @pl.when(b < NCH)
        def _():
            pltpu.make_async_copy(src_hbm.at[wid, b], srcb2, sib).wait()
            pltpu.async_copy(x_hbm.at[srcb2], rowsb, sgb)
        pltpu.make_async_copy(x_hbm.at[srca], rowsa, sga).wait()

        @pl.when(a + 2 < NCH)
        def _():
            pltpu.async_copy(src_hbm.at[wid, a + 2], srca, sia)
        pltpu.async_copy(rowsa, agg_sh.at[dstb.at[a]], ssa, add=True)
        pltpu.sync_copy(onesb, deg_sh.at[dstb.at[a]], add=True)

        @pl.when(b < NCH)
        def _():
            pltpu.make_async_copy(x_hbm.at[srcb2], rowsb, sgb).wait()

        @pl.when(b + 2 < NCH)
        def _():
            pltpu.async_copy(src_hbm.at[wid, b + 2], srcb2, sib)

        @pl.when(b < NCH)
        def _():
            pltpu.async_copy(rowsb, agg_sh.at[dstb.at[b]], ssb, add=True)
            pltpu.sync_copy(onesb, deg_sh.at[dstb.at[b]], add=True)

        pltpu.make_async_copy(rowsa, agg_sh.at[dstb.at[a]], ssa).wait()

        @pl.when(a + 2 < NCH)
        def _():
            pltpu.make_async_copy(src_hbm.at[wid, a + 2], srca, sia).wait()
            pltpu.async_copy(x_hbm.at[srca], rowsa, sga)

        @pl.when(b < NCH)
        def _():
            pltpu.make_async_copy(rowsb, agg_sh.at[dstb.at[b]], ssb).wait()
        return carry

    lax.fori_loop(0, (NCH + 1) // 2, body, 0)
    plsc.subcore_barrier()
    for t in range(NZ):
        pltpu.sync_copy(agg_sh.at[pl.ds(s * NPT + t * CH, CH)], rowsa)
        pltpu.sync_copy(rowsa, agg_out.at[c, pl.ds(s * NPT + t * CH, CH)])
    pltpu.sync_copy(deg_sh.at[pl.ds(s * NPT, NPT)], stgd)
    pltpu.sync_copy(stgd, deg_out.at[c, pl.ds(s * NPT, NPT)])


@functools.partial(
    pl.kernel,
    mesh=_mesh,
    out_type=jax.ShapeDtypeStruct((NC, NP_), jnp.float32),
    scratch_types=[
        pltpu.VMEM((NCH, CH), jnp.int32),
        pltpu.VMEM((NCH, CH), jnp.int32),
        [pltpu.VMEM((CH,), jnp.float32)] * 4,
        pltpu.VMEM((NPT,), jnp.float32),
        pltpu.VMEM_SHARED((NP_,), jnp.float32),
        [pltpu.SemaphoreType.DMA] * 4,
        [pltpu.SemaphoreType.DMA] * 4,
    ],
)
def _sc_pass2(hn_hbm, src_hbm, dst_hbm, zdeg_hbm,
              acc_out, srcb, dstb, vals, stgd, acc_sh, sg, ss):
    c = lax.axis_index("c")
    s = lax.axis_index("s")
    wid = c * NS + s
    pltpu.sync_copy(zdeg_hbm, stgd)
    pltpu.sync_copy(src_hbm.at[wid], srcb)
    pltpu.sync_copy(dst_hbm.at[wid], dstb)
    pltpu.sync_copy(stgd, acc_sh.at[pl.ds(s * NPT, NPT)])
    for q in range(4):
        pltpu.async_copy(hn_hbm.at[srcb.at[q]], vals[q], sg[q])
    plsc.subcore_barrier()

    def body(t, carry):
        for q in range(4):
            ch = 4 * t + q

            @pl.when(ch < NCH)
            def _(q=q, ch=ch):
                pltpu.make_async_copy(hn_hbm.at[srcb.at[ch]], vals[q], sg[q]).wait()
                pltpu.async_copy(vals[q], acc_sh.at[dstb.at[ch]], ss[q], add=True)
        for q in range(4):
            ch = 4 * t + 4 + q

            @pl.when(ch < NCH)
            def _(q=q, ch=ch):
                pltpu.make_async_copy(vals[q], acc_sh.at[dstb.at[ch - 4]], ss[q]).wait()
                pltpu.async_copy(hn_hbm.at[srcb.at[ch]], vals[q], sg[q])
        return carry

    lax.fori_loop(0, (NCH + 3) // 4, body, 0)
    for q in range(4):
        last = ((NCH - 1 - q) // 4) * 4 + q
        pltpu.make_async_copy(vals[q], acc_sh.at[dstb.at[last]], ss[q]).wait()
    plsc.subcore_barrier()
    pltpu.sync_copy(acc_sh.at[pl.ds(s * NPT, NPT)], stgd)
    pltpu.sync_copy(stgd, acc_out.at[c, pl.ds(s * NPT, NPT)])


def _tc_a_body(x_ref, aggp_ref, degp_ref, ws_ref, wn_ref, b_ref,
               wscn_ref, wscs_ref, h_ref, hn_ref, hs_ref):
    x = x_ref[...]
    agg = aggp_ref[0] + aggp_ref[1]
    deg = jnp.maximum(degp_ref[0] + degp_ref[1], 1.0)
    m = agg / deg
    h = jnp.maximum(
        jnp.dot(x, ws_ref[...], preferred_element_type=jnp.float32)
        + jnp.dot(m, wn_ref[...], preferred_element_type=jnp.float32)
        + b_ref[...], 0.0)
    h_ref[...] = h
    hn_ref[...] = jnp.sum(h * wscn_ref[...], axis=1, keepdims=True)
    hs_ref[...] = jnp.sum(h * wscs_ref[...], axis=1, keepdims=True)


def _tc_a(x, aggp, degp3, ws, wn, b, wscn, wscs):
    blk = 1000
    grid = N // blk
    return pl.pallas_call(
        _tc_a_body,
        grid=(grid,),
        in_specs=[
            pl.BlockSpec((blk, D), lambda i: (i, 0)),
            pl.BlockSpec((NC, blk, D), lambda i: (0, i, 0)),
            pl.BlockSpec((NC, blk, 1), lambda i: (0, i, 0)),
            pl.BlockSpec((D, D), lambda i: (0, 0)),
            pl.BlockSpec((D, D), lambda i: (0, 0)),
            pl.BlockSpec((1, D), lambda i: (0, 0)),
            pl.BlockSpec((1, D), lambda i: (0, 0)),
            pl.BlockSpec((1, D), lambda i: (0, 0)),
        ],
        out_specs=[
            pl.BlockSpec((blk, D), lambda i: (i, 0)),
            pl.BlockSpec((blk, 1), lambda i: (i, 0)),
            pl.BlockSpec((blk, 1), lambda i: (i, 0)),
        ],
        out_shape=[
            jax.ShapeDtypeStruct((N, D), jnp.float32),
            jax.ShapeDtypeStruct((N, 1), jnp.float32),
            jax.ShapeDtypeStruct((N, 1), jnp.float32),
        ],
    )(x, aggp, degp3, ws, wn, b, wscn, wscs)


def _tc_b_body(h_ref, hs_ref, aggsp_ref, bsc_ref, gamma_ref, beta_ref,
               wfc_ref, bfc_ref, out_ref):
    hs = hs_ref[...]
    aggs = aggsp_ref[0] + aggsp_ref[1]
    sg = hs + aggs + bsc_ref[0, 0]
    iota = lax.broadcasted_iota(jnp.int32, (G, PER), 1)
    sel = jnp.zeros((G, PER), jnp.bool_)
    for _ in range(K):
        masked = jnp.where(sel, -jnp.inf, sg)
        mx = jnp.max(masked, axis=1, keepdims=True)
        pos = jnp.min(jnp.where(masked == mx, iota, PER), axis=1, keepdims=True)
        sel = jnp.logical_or(sel, iota == pos)
    w = jnp.where(sel, jnp.tanh(sg), 0.0)
    h = h_ref[...]
    S = jnp.zeros((G, D), jnp.float32)
    Q = jnp.zeros((G, D), jnp.float32)
    w2 = w * w
    for j in range(PER):
        hj = h[:, j, :]
        S = S + w[:, j:j + 1] * hj
        Q = Q + w2[:, j:j + 1] * (hj * hj)
    tot = jnp.float32(G * K)
    mu = jnp.sum(S, axis=0, keepdims=True) / tot
    q2 = jnp.sum(Q, axis=0, keepdims=True) / tot
    var = q2 - mu * mu
    rstd = lax.rsqrt(var + 1e-5)
    pooled = (S / jnp.float32(K) - mu) * rstd * gamma_ref[...] + beta_ref[...]
    logits = jnp.dot(pooled, wfc_ref[...], preferred_element_type=jnp.float32) + bfc_ref[...]
    lmx = jnp.max(logits, axis=1, keepdims=True)
    lse = jnp.log(jnp.sum(jnp.exp(logits - lmx), axis=1, keepdims=True)) + lmx
    out_ref[...] = logits - lse


def _tc_b(h3, hs2, aggsp, bsc, gamma, beta, wfc, bfc):
    return pl.pallas_call(
        _tc_b_body,
        out_shape=jax.ShapeDtypeStruct((G, OUT), jnp.float32),
    )(h3, hs2, aggsp, bsc, gamma, beta, wfc, bfc)


def kernel(x, edge_index, batch, W_enc_self, W_enc_nbr, b_enc,
           W_sc_self, W_sc_nbr, b_sc, gamma, beta, W_fc, b_fc):
    srcw = edge_index[0].astype(jnp.int32).reshape(NW, EPW)
    dstw = edge_index[1].astype(jnp.int32).reshape(NW, EPW)
    pad_src = ((jnp.arange(NW * PAD, dtype=jnp.int32) * 97) % N).reshape(NW, PAD)
    pad_dst = (N + (jnp.arange(NW * PAD, dtype=jnp.int32) % (NP_ - N))).reshape(NW, PAD)
    src3 = jnp.concatenate([srcw, pad_src], axis=1).reshape(NW, NCH, CH)
    dst3 = jnp.concatenate([dstw, pad_dst], axis=1).reshape(NW, NCH, CH)
    z128 = jnp.zeros((CH, D), jnp.float32)
    zdeg = jnp.zeros((NPT,), jnp.float32)
    ones = jnp.ones((CH,), jnp.float32)

    aggp, degp = _sc_pass1(x, src3, dst3, z128, zdeg, ones)
    h, hn, hs = _tc_a(x, aggp, degp.reshape(NC, NP_, 1)[:, :N, :],
                      W_enc_self, W_enc_nbr, b_enc.reshape(1, D),
                      W_sc_nbr.reshape(1, D), W_sc_self.reshape(1, D))
    aggsp = _sc_pass2(hn.reshape(N), src3, dst3, zdeg)
    out = _tc_b(h.reshape(G, PER, D), hs.reshape(G, PER),
                aggsp[:, :N].reshape(NC, G, PER), b_sc.reshape(1, 1),
                gamma.reshape(1, D), beta.reshape(1, D), W_fc,
                b_fc.reshape(1, OUT))
    return out


# speedup vs baseline: 1.1446x; 1.1446x over previous
import functools

import jax
import jax.numpy as jnp
from jax import lax
from jax.experimental import pallas as pl
from jax.experimental.pallas import tpu as pltpu
from jax.experimental.pallas import tpu_sc as plsc

N = 10000
E = 320000
G = 200
D = 128
OUT = 4
PER = N // G
K = PER // 2
NC = 2
NS = 16
NW = NC * NS
EPW = E // NW
CH = 128
NCH = 79
PAD = NCH * CH - EPW
NP_ = 10240
NPT = NP_ // NS
NZ = NPT // CH

_mesh = plsc.VectorSubcoreMesh(core_axis_name="c", subcore_axis_name="s")


@functools.partial(
    pl.kernel,
    mesh=_mesh,
    out_type=[
        jax.ShapeDtypeStruct((NC, NP_, D), jnp.float32),
        jax.ShapeDtypeStruct((NC, NP_), jnp.float32),
    ],
    scratch_types=[
        pltpu.VMEM((CH,), jnp.int32),
        pltpu.VMEM((CH,), jnp.int32),
        pltpu.VMEM((NCH, CH), jnp.int32),
        pltpu.VMEM((CH, D), jnp.float32),
        pltpu.VMEM((CH, D), jnp.float32),
        pltpu.VMEM((CH,), jnp.float32),
        pltpu.VMEM((NPT,), jnp.float32),
        pltpu.VMEM_SHARED((NP_, D), jnp.float32),
        pltpu.VMEM_SHARED((NP_,), jnp.float32),
        pltpu.SemaphoreType.DMA,
        pltpu.SemaphoreType.DMA,
        pltpu.SemaphoreType.DMA,
        pltpu.SemaphoreType.DMA,
        pltpu.SemaphoreType.DMA,
        pltpu.SemaphoreType.DMA,
    ],
)
def _sc_pass1(x_hbm, src_hbm, dst_hbm, z128_hbm, zdeg_hbm, ones_hbm,
              agg_out, deg_out, srca, srcb2, dstb, rowsa, rowsb, onesb, stgd,
              agg_sh, deg_sh, sga, sgb, sia, sib, ssa, ssb):
    c = lax.axis_index("c")
    s = lax.axis_index("s")
    wid = c * NS + s
    pltpu.sync_copy(z128_hbm, rowsa)
    pltpu.sync_copy(zdeg_hbm, stgd)
    pltpu.sync_copy(ones_hbm, onesb)
    pltpu.sync_copy(dst_hbm.at[wid], dstb)
    for t in range(NZ):
        pltpu.sync_copy(rowsa, agg_sh.at[pl.ds(s * NPT + t * CH, CH)])
    pltpu.sync_copy(stgd, deg_sh.at[pl.ds(s * NPT, NPT)])
    pltpu.sync_copy(src_hbm.at[wid, 0], srca)
    pltpu.async_copy(x_hbm.at[srca], rowsa, sga)
    pltpu.async_copy(src_hbm.at[wid, 1], srcb2, sib)
    plsc.subcore_barrier()

    def body(t, carry):
        a = 2 * t
        b = a + 1

        @pl.when(b < NCH)
        def _():
            pltpu.make_async_copy(src_hbm.at[wid, b], srcb2, sib).wait()
            pltpu.async_copy(x_hbm.at[srcb2], rowsb, sgb)
        pltpu.make_async_copy(x_hbm.at[srca], rowsa, sga).wait()

        @pl.when(a + 2 < NCH)
        def _():
            pltpu.async_copy(src_hbm.at[wid, a + 2], srca, sia)
        pltpu.sync_copy(rowsa, agg_sh.at[dstb.at[a]], add=True)
        pltpu.sync_copy(onesb, deg_sh.at[dstb.at[a]], add=True)

        @pl.when(a + 2 < NCH)
        def _():
            pltpu.make_async_copy(src_hbm.at[wid, a + 2], srca, sia).wait()
            pltpu.async_copy(x_hbm.at[srca], rowsa, sga)

        @pl.when(b < NCH)
        def _():
            pltpu.make_async_copy(x_hbm.at[srcb2], rowsb, sgb).wait()

        @pl.when(b + 2 < NCH)
        def _():
            pltpu.async_copy(src_hbm.at[wid, b + 2], srcb2, sib)

        @pl.when(b < NCH)
        def _():
            pltpu.sync_copy(rowsb, agg_sh.at[dstb.at[b]], add=True)
            pltpu.sync_copy(onesb, deg_sh.at[dstb.at[b]], add=True)
        return carry

    lax.fori_loop(0, (NCH + 1) // 2, body, 0)
    plsc.subcore_barrier()
    for t in range(NZ):
        pltpu.sync_copy(agg_sh.at[pl.ds(s * NPT + t * CH, CH)], rowsa)
        pltpu.sync_copy(rowsa, agg_out.at[c, pl.ds(s * NPT + t * CH, CH)])
    pltpu.sync_copy(deg_sh.at[pl.ds(s * NPT, NPT)], stgd)
    pltpu.sync_copy(stgd, deg_out.at[c, pl.ds(s * NPT, NPT)])


@functools.partial(
    pl.kernel,
    mesh=_mesh,
    out_type=jax.ShapeDtypeStruct((NC, NP_), jnp.float32),
    scratch_types=[
        pltpu.VMEM((NCH, CH), jnp.int32),
        pltpu.VMEM((NCH, CH), jnp.int32),
        [pltpu.VMEM((CH,), jnp.float32)] * 4,
        pltpu.VMEM((NPT,), jnp.float32),
        pltpu.VMEM_SHARED((NP_,), jnp.float32),
        [pltpu.SemaphoreType.DMA] * 4,
        [pltpu.SemaphoreType.DMA] * 4,
    ],
)
def _sc_pass2(hn_hbm, src_hbm, dst_hbm, zdeg_hbm,
              acc_out, srcb, dstb, vals, stgd, acc_sh, sg, ss):
    c = lax.axis_index("c")
    s = lax.axis_index("s")
    wid = c * NS + s
    pltpu.sync_copy(zdeg_hbm, stgd)
    pltpu.sync_copy(src_hbm.at[wid], srcb)
    pltpu.sync_copy(dst_hbm.at[wid], dstb)
    pltpu.sync_copy(stgd, acc_sh.at[pl.ds(s * NPT, NPT)])
    for q in range(4):
        pltpu.async_copy(hn_hbm.at[srcb.at[q]], vals[q], sg[q])
    plsc.subcore_barrier()

    def body(t, carry):
        for q in range(4):
            ch = 4 * t + q

            @pl.when(ch < NCH)
            def _(q=q, ch=ch):
                pltpu.make_async_copy(hn_hbm.at[srcb.at[ch]], vals[q], sg[q]).wait()
                pltpu.async_copy(vals[q], acc_sh.at[dstb.at[ch]], ss[q], add=True)
        for q in range(4):
            ch = 4 * t + 4 + q

            @pl.when(ch < NCH)
            def _(q=q, ch=ch):
                pltpu.make_async_copy(vals[q], acc_sh.at[dstb.at[ch - 4]], ss[q]).wait()
                pltpu.async_copy(hn_hbm.at[srcb.at[ch]], vals[q], sg[q])
        return carry

    lax.fori_loop(0, (NCH + 3) // 4, body, 0)
    for q in range(4):
        last = ((NCH - 1 - q) // 4) * 4 + q
        pltpu.make_async_copy(vals[q], acc_sh.at[dstb.at[last]], ss[q]).wait()
    plsc.subcore_barrier()
    pltpu.sync_copy(acc_sh.at[pl.ds(s * NPT, NPT)], stgd)
    pltpu.sync_copy(stgd, acc_out.at[c, pl.ds(s * NPT, NPT)])


def _tc_a_body(x_ref, aggp_ref, degp_ref, ws_ref, wn_ref, b_ref,
               wscn_ref, wscs_ref, h_ref, hn_ref, hs_ref):
    x = x_ref[...]
    agg = aggp_ref[0] + aggp_ref[1]
    deg = jnp.maximum(degp_ref[0] + degp_ref[1], 1.0)
    m = agg / deg
    h = jnp.maximum(
        jnp.dot(x, ws_ref[...], preferred_element_type=jnp.float32)
        + jnp.dot(m, wn_ref[...], preferred_element_type=jnp.float32)
        + b_ref[...], 0.0)
    h_ref[...] = h
    hn_ref[...] = jnp.sum(h * wscn_ref[...], axis=1, keepdims=True)
    hs_ref[...] = jnp.sum(h * wscs_ref[...], axis=1, keepdims=True)


def _tc_a(x, aggp, degp3, ws, wn, b, wscn, wscs):
    blk = 1000
    grid = N // blk
    return pl.pallas_call(
        _tc_a_body,
        grid=(grid,),
        in_specs=[
            pl.BlockSpec((blk, D), lambda i: (i, 0)),
            pl.BlockSpec((NC, blk, D), lambda i: (0, i, 0)),
            pl.BlockSpec((NC, blk, 1), lambda i: (0, i, 0)),
            pl.BlockSpec((D, D), lambda i: (0, 0)),
            pl.BlockSpec((D, D), lambda i: (0, 0)),
            pl.BlockSpec((1, D), lambda i: (0, 0)),
            pl.BlockSpec((1, D), lambda i: (0, 0)),
            pl.BlockSpec((1, D), lambda i: (0, 0)),
        ],
        out_specs=[
            pl.BlockSpec((blk, D), lambda i: (i, 0)),
            pl.BlockSpec((blk, 1), lambda i: (i, 0)),
            pl.BlockSpec((blk, 1), lambda i: (i, 0)),
        ],
        out_shape=[
            jax.ShapeDtypeStruct((N, D), jnp.float32),
            jax.ShapeDtypeStruct((N, 1), jnp.float32),
            jax.ShapeDtypeStruct((N, 1), jnp.float32),
        ],
    )(x, aggp, degp3, ws, wn, b, wscn, wscs)


def _tc_b_body(h_ref, hs_ref, aggsp_ref, bsc_ref, gamma_ref, beta_ref,
               wfc_ref, bfc_ref, out_ref):
    hs = hs_ref[...]
    aggs = aggsp_ref[0] + aggsp_ref[1]
    sg = hs + aggs + bsc_ref[0, 0]
    iota = lax.broadcasted_iota(jnp.int32, (G, PER), 1)
    sel = jnp.zeros((G, PER), jnp.bool_)
    for _ in range(K):
        masked = jnp.where(sel, -jnp.inf, sg)
        mx = jnp.max(masked, axis=1, keepdims=True)
        pos = jnp.min(jnp.where(masked == mx, iota, PER), axis=1, keepdims=True)
        sel = jnp.logical_or(sel, iota == pos)
    w = jnp.where(sel, jnp.tanh(sg), 0.0)
    h = h_ref[...]
    S = jnp.zeros((G, D), jnp.float32)
    Q = jnp.zeros((G, D), jnp.float32)
    w2 = w * w
    for j in range(PER):
        hj = h[:, j, :]
        S = S + w[:, j:j + 1] * hj
        Q = Q + w2[:, j:j + 1] * (hj * hj)
    tot = jnp.float32(G * K)
    mu = jnp.sum(S, axis=0, keepdims=True) / tot
    q2 = jnp.sum(Q, axis=0, keepdims=True) / tot
    var = q2 - mu * mu
    rstd = lax.rsqrt(var + 1e-5)
    pooled = (S / jnp.float32(K) - mu) * rstd * gamma_ref[...] + beta_ref[...]
    logits = jnp.dot(pooled, wfc_ref[...], preferred_element_type=jnp.float32) + bfc_ref[...]
    lmx = jnp.max(logits, axis=1, keepdims=True)
    lse = jnp.log(jnp.sum(jnp.exp(logits - lmx), axis=1, keepdims=True)) + lmx
    out_ref[...] = logits - lse


def _tc_b(h3, hs2, aggsp, bsc, gamma, beta, wfc, bfc):
    return pl.pallas_call(
        _tc_b_body,
        out_shape=jax.ShapeDtypeStruct((G, OUT), jnp.float32),
    )(h3, hs2, aggsp, bsc, gamma, beta, wfc, bfc)


def kernel(x, edge_index, batch, W_enc_self, W_enc_nbr, b_enc,
           W_sc_self, W_sc_nbr, b_sc, gamma, beta, W_fc, b_fc):
    srcw = edge_index[0].astype(jnp.int32).reshape(NW, EPW)
    dstw = edge_index[1].astype(jnp.int32).reshape(NW, EPW)
    pad_src = ((jnp.arange(NW * PAD, dtype=jnp.int32) * 97) % N).reshape(NW, PAD)
    pad_dst = (N + (jnp.arange(NW * PAD, dtype=jnp.int32) % (NP_ - N))).reshape(NW, PAD)
    src3 = jnp.concatenate([srcw, pad_src], axis=1).reshape(NW, NCH, CH)
    dst3 = jnp.concatenate([dstw, pad_dst], axis=1).reshape(NW, NCH, CH)
    z128 = jnp.zeros((CH, D), jnp.float32)
    zdeg = jnp.zeros((NPT,), jnp.float32)
    ones = jnp.ones((CH,), jnp.float32)

    aggp, degp = _sc_pass1(x, src3, dst3, z128, zdeg, ones)
    h, hn, hs = _tc_a(x, aggp, degp.reshape(NC, NP_, 1)[:, :N, :],
                      W_enc_self, W_enc_nbr, b_enc.reshape(1, D),
                      W_sc_nbr.reshape(1, D), W_sc_self.reshape(1, D))
    aggsp = _sc_pass2(hn.reshape(N), src3, dst3, zdeg)
    out = _tc_b(h.reshape(G, PER, D), hs.reshape(G, PER),
                aggsp[:, :N].reshape(NC, G, PER), b_sc.reshape(1, 1),
                gamma.reshape(1, D), beta.reshape(1, D), W_fc,
                b_fc.reshape(1, OUT))
    return out


# speedup vs baseline: 1.3443x; 1.1745x over previous
import functools

import jax
import jax.numpy as jnp
from jax import lax
from jax.experimental import pallas as pl
from jax.experimental.pallas import tpu as pltpu
from jax.experimental.pallas import tpu_sc as plsc

N = 10000
E = 320000
G = 200
D = 128
OUT = 4
PER = N // G
K = PER // 2
NC = 2
NS = 16
NW = NC * NS
EPW = E // NW
CH = 128
NCH = 79
PAD = NCH * CH - EPW
NP_ = 10240
NPT = NP_ // NS
NZ = NPT // CH

_mesh = plsc.VectorSubcoreMesh(core_axis_name="c", subcore_axis_name="s")


@functools.partial(
    pl.kernel,
    mesh=_mesh,
    out_type=[
        jax.ShapeDtypeStruct((NC, NP_, D), jnp.float32),
        jax.ShapeDtypeStruct((NC, NP_), jnp.float32),
    ],
    scratch_types=[
        pltpu.VMEM((CH,), jnp.int32),
        pltpu.VMEM((CH,), jnp.int32),
        pltpu.VMEM((NCH, CH), jnp.int32),
        pltpu.VMEM((CH, D), jnp.float32),
        pltpu.VMEM((CH, D), jnp.float32),
        pltpu.VMEM((CH,), jnp.float32),
        pltpu.VMEM((NPT,), jnp.float32),
        pltpu.VMEM_SHARED((NP_, D), jnp.float32),
        pltpu.VMEM_SHARED((NP_,), jnp.float32),
        pltpu.SemaphoreType.DMA,
        pltpu.SemaphoreType.DMA,
        pltpu.SemaphoreType.DMA,
        pltpu.SemaphoreType.DMA,
        pltpu.SemaphoreType.DMA,
        pltpu.SemaphoreType.DMA,
    ],
)
def _sc_pass1(x_hbm, src_hbm, dst_hbm, z128_hbm, zdeg_hbm, ones_hbm,
              agg_out, deg_out, srca, srcb2, dstb, rowsa, rowsb, onesb, stgd,
              agg_sh, deg_sh, sga, sgb, sia, sib, ssa, ssb):
    c = lax.axis_index("c")
    s = lax.axis_index("s")
    wid = c * NS + s
    pltpu.sync_copy(z128_hbm, rowsa)
    pltpu.sync_copy(zdeg_hbm, stgd)
    pltpu.sync_copy(ones_hbm, onesb)
    pltpu.sync_copy(dst_hbm.at[wid], dstb)
    for t in range(NZ):
        pltpu.sync_copy(rowsa, agg_sh.at[pl.ds(s * NPT + t * CH, CH)])
    pltpu.sync_copy(stgd, deg_sh.at[pl.ds(s * NPT, NPT)])
    pltpu.sync_copy(src_hbm.at[wid, 0], srca)
    pltpu.async_copy(x_hbm.at[srca], rowsa, sga)
    pltpu.async_copy(src_hbm.at[wid, 1], srcb2, sib)
    plsc.subcore_barrier()

    def body(t, carry):
        a = 2 * t
        b = a + 1

        @pl.when(b < NCH)
        def _():
            pltpu.make_async_copy(src_hbm.at[wid, b], srcb2, sib).wait()
            pltpu.async_copy(x_hbm.at[srcb2], rowsb, sgb)
        pltpu.make_async_copy(x_hbm.at[srca], rowsa, sga).wait()

        @pl.when(a + 2 < NCH)
        def _():
            pltpu.async_copy(src_hbm.at[wid, a + 2], srca, sia)
        pltpu.sync_copy(rowsa, agg_sh.at[dstb.at[a]], add=True)
        pltpu.sync_copy(onesb, deg_sh.at[dstb.at[a]], add=True)

        @pl.when(a + 2 < NCH)
        def _():
            pltpu.make_async_copy(src_hbm.at[wid, a + 2], srca, sia).wait()
            pltpu.async_copy(x_hbm.at[srca], rowsa, sga)

        @pl.when(b < NCH)
        def _():
            pltpu.make_async_copy(x_hbm.at[srcb2], rowsb, sgb).wait()

        @pl.when(b + 2 < NCH)
        def _():
            pltpu.async_copy(src_hbm.at[wid, b + 2], srcb2, sib)

        @pl.when(b < NCH)
        def _():
            pltpu.sync_copy(rowsb, agg_sh.at[dstb.at[b]], add=True)
            pltpu.sync_copy(onesb, deg_sh.at[dstb.at[b]], add=True)
        return carry

    lax.fori_loop(0, (NCH + 1) // 2, body, 0)
    plsc.subcore_barrier()
    for t in range(NZ):
        pltpu.sync_copy(agg_sh.at[pl.ds(s * NPT + t * CH, CH)], rowsa)
        pltpu.sync_copy(rowsa, agg_out.at[c, pl.ds(s * NPT + t * CH, CH)])
    pltpu.sync_copy(deg_sh.at[pl.ds(s * NPT, NPT)], stgd)
    pltpu.sync_copy(stgd, deg_out.at[c, pl.ds(s * NPT, NPT)])


@functools.partial(
    pl.kernel,
    mesh=_mesh,
    out_type=jax.ShapeDtypeStruct((NC, NP_), jnp.float32),
    compiler_params=pltpu.CompilerParams(needs_layout_passes=False),
    scratch_types=[
        pltpu.VMEM((N,), jnp.float32),
        pltpu.VMEM((NCH, CH), jnp.int32),
        pltpu.VMEM((NCH, CH), jnp.int32),
        [pltpu.VMEM((CH,), jnp.float32)] * 2,
        pltpu.VMEM((NPT,), jnp.float32),
        pltpu.VMEM_SHARED((NP_,), jnp.float32),
        [pltpu.SemaphoreType.DMA] * 2,
    ],
)
def _sc_pass2(hn_hbm, src_hbm, dst_hbm, zdeg_hbm,
              acc_out, hn_v, srcb, dstb, vals, stgd, acc_sh, ss):
    c = lax.axis_index("c")
    s = lax.axis_index("s")
    wid = c * NS + s
    pltpu.sync_copy(zdeg_hbm, stgd)
    pltpu.sync_copy(hn_hbm, hn_v)
    pltpu.sync_copy(src_hbm.at[wid], srcb)
    pltpu.sync_copy(dst_hbm.at[wid], dstb)
    pltpu.sync_copy(stgd, acc_sh.at[pl.ds(s * NPT, NPT)])
    plsc.subcore_barrier()

    def fill(q, ch):
        for kk in range(CH // 16):
            s16 = srcb[ch, pl.ds(kk * 16, 16)]
            g = plsc.load_gather(hn_v, [s16])
            vals[q][pl.ds(kk * 16, 16)] = g

    def body(t, carry):
        for q in range(2):
            ch = 2 * t + q

            @pl.when(ch < NCH)
            def _(q=q, ch=ch):
                @pl.when(ch >= 2)
                def _():
                    pltpu.make_async_copy(vals[q], acc_sh.at[dstb.at[ch - 2]], ss[q]).wait()
                fill(q, ch)
                pltpu.async_copy(vals[q], acc_sh.at[dstb.at[ch]], ss[q], add=True)
        return carry

    lax.fori_loop(0, (NCH + 1) // 2, body, 0)
    for q in range(2):
        last = ((NCH - 1 - q) // 2) * 2 + q
        pltpu.make_async_copy(vals[q], acc_sh.at[dstb.at[last]], ss[q]).wait()
    plsc.subcore_barrier()
    pltpu.sync_copy(acc_sh.at[pl.ds(s * NPT, NPT)], stgd)
    pltpu.sync_copy(stgd, acc_out.at[c, pl.ds(s * NPT, NPT)])


def _tc_a_body(x_ref, aggp_ref, degp_ref, ws_ref, wn_ref, b_ref,
               wscn_ref, wscs_ref, h_ref, hn_ref, hs_ref):
    x = x_ref[...]
    agg = aggp_ref[0] + aggp_ref[1]
    deg = jnp.maximum(degp_ref[0] + degp_ref[1], 1.0)
    m = agg / deg
    h = jnp.maximum(
        jnp.dot(x, ws_ref[...], preferred_element_type=jnp.float32)
        + jnp.dot(m, wn_ref[...], preferred_element_type=jnp.float32)
        + b_ref[...], 0.0)
    h_ref[...] = h
    hn_ref[...] = jnp.sum(h * wscn_ref[...], axis=1, keepdims=True)
    hs_ref[...] = jnp.sum(h * wscs_ref[...], axis=1, keepdims=True)


def _tc_a(x, aggp, degp3, ws, wn, b, wscn, wscs):
    blk = 1000
    grid = N // blk
    return pl.pallas_call(
        _tc_a_body,
        grid=(grid,),
        in_specs=[
            pl.BlockSpec((blk, D), lambda i: (i, 0)),
            pl.BlockSpec((NC, blk, D), lambda i: (0, i, 0)),
            pl.BlockSpec((NC, blk, 1), lambda i: (0, i, 0)),
            pl.BlockSpec((D, D), lambda i: (0, 0)),
            pl.BlockSpec((D, D), lambda i: (0, 0)),
            pl.BlockSpec((1, D), lambda i: (0, 0)),
            pl.BlockSpec((1, D), lambda i: (0, 0)),
            pl.BlockSpec((1, D), lambda i: (0, 0)),
        ],
        out_specs=[
            pl.BlockSpec((blk, D), lambda i: (i, 0)),
            pl.BlockSpec((blk, 1), lambda i: (i, 0)),
            pl.BlockSpec((blk, 1), lambda i: (i, 0)),
        ],
        out_shape=[
            jax.ShapeDtypeStruct((N, D), jnp.float32),
            jax.ShapeDtypeStruct((N, 1), jnp.float32),
            jax.ShapeDtypeStruct((N, 1), jnp.float32),
        ],
    )(x, aggp, degp3, ws, wn, b, wscn, wscs)


def _tc_b_body(h_ref, hs_ref, aggsp_ref, bsc_ref, gamma_ref, beta_ref,
               wfc_ref, bfc_ref, out_ref):
    hs = hs_ref[...]
    aggs = aggsp_ref[0] + aggsp_ref[1]
    sg = hs + aggs + bsc_ref[0, 0]
    iota = lax.broadcasted_iota(jnp.int32, (G, PER), 1)
    sel = jnp.zeros((G, PER), jnp.bool_)
    for _ in range(K):
        masked = jnp.where(sel, -jnp.inf, sg)
        mx = jnp.max(masked, axis=1, keepdims=True)
        pos = jnp.min(jnp.where(masked == mx, iota, PER), axis=1, keepdims=True)
        sel = jnp.logical_or(sel, iota == pos)
    w = jnp.where(sel, jnp.tanh(sg), 0.0)
    h = h_ref[...]
    S = jnp.zeros((G, D), jnp.float32)
    Q = jnp.zeros((G, D), jnp.float32)
    w2 = w * w
    for j in range(PER):
        hj = h[:, j, :]
        S = S + w[:, j:j + 1] * hj
        Q = Q + w2[:, j:j + 1] * (hj * hj)
    tot = jnp.float32(G * K)
    mu = jnp.sum(S, axis=0, keepdims=True) / tot
    q2 = jnp.sum(Q, axis=0, keepdims=True) / tot
    var = q2 - mu * mu
    rstd = lax.rsqrt(var + 1e-5)
    pooled = (S / jnp.float32(K) - mu) * rstd * gamma_ref[...] + beta_ref[...]
    logits = jnp.dot(pooled, wfc_ref[...], preferred_element_type=jnp.float32) + bfc_ref[...]
    lmx = jnp.max(logits, axis=1, keepdims=True)
    lse = jnp.log(jnp.sum(jnp.exp(logits - lmx), axis=1, keepdims=True)) + lmx
    out_ref[...] = logits - lse


def _tc_b(h3, hs2, aggsp, bsc, gamma, beta, wfc, bfc):
    return pl.pallas_call(
        _tc_b_body,
        out_shape=jax.ShapeDtypeStruct((G, OUT), jnp.float32),
    )(h3, hs2, aggsp, bsc, gamma, beta, wfc, bfc)


def kernel(x, edge_index, batch, W_enc_self, W_enc_nbr, b_enc,
           W_sc_self, W_sc_nbr, b_sc, gamma, beta, W_fc, b_fc):
    srcw = edge_index[0].astype(jnp.int32).reshape(NW, EPW)
    dstw = edge_index[1].astype(jnp.int32).reshape(NW, EPW)
    pad_src = ((jnp.arange(NW * PAD, dtype=jnp.int32) * 97) % N).reshape(NW, PAD)
    pad_dst = (N + (jnp.arange(NW * PAD, dtype=jnp.int32) % (NP_ - N))).reshape(NW, PAD)
    src3 = jnp.concatenate([srcw, pad_src], axis=1).reshape(NW, NCH, CH)
    dst3 = jnp.concatenate([dstw, pad_dst], axis=1).reshape(NW, NCH, CH)
    z128 = jnp.zeros((CH, D), jnp.float32)
    zdeg = jnp.zeros((NPT,), jnp.float32)
    ones = jnp.ones((CH,), jnp.float32)

    aggp, degp = _sc_pass1(x, src3, dst3, z128, zdeg, ones)
    h, hn, hs = _tc_a(x, aggp, degp.reshape(NC, NP_, 1)[:, :N, :],
                      W_enc_self, W_enc_nbr, b_enc.reshape(1, D),
                      W_sc_nbr.reshape(1, D), W_sc_self.reshape(1, D))
    aggsp = _sc_pass2(hn.reshape(N), src3, dst3, zdeg)
    out = _tc_b(h.reshape(G, PER, D), hs.reshape(G, PER),
                aggsp[:, :N].reshape(NC, G, PER), b_sc.reshape(1, 1),
                gamma.reshape(1, D), beta.reshape(1, D), W_fc,
                b_fc.reshape(1, OUT))
    return out


# speedup vs baseline: 1.3535x; 1.0069x over previous
import functools

import jax
import jax.numpy as jnp
from jax import lax
from jax.experimental import pallas as pl
from jax.experimental.pallas import tpu as pltpu
from jax.experimental.pallas import tpu_sc as plsc

N = 10000
E = 320000
G = 200
D = 128
OUT = 4
PER = N // G
K = PER // 2
NC = 2
NS = 16
NW = NC * NS
EPW = E // NW
CH = 128
NCH = 79
PAD = NCH * CH - EPW
NP_ = 10240
NPT = NP_ // NS
NZ = NPT // CH

_mesh = plsc.VectorSubcoreMesh(core_axis_name="c", subcore_axis_name="s")


@functools.partial(
    pl.kernel,
    mesh=_mesh,
    out_type=[
        jax.ShapeDtypeStruct((NC, NP_, D), jnp.float32),
        jax.ShapeDtypeStruct((NC, NP_), jnp.float32),
    ],
    scratch_types=[
        pltpu.VMEM((CH,), jnp.int32),
        pltpu.VMEM((CH,), jnp.int32),
        pltpu.VMEM((NCH, CH), jnp.int32),
        pltpu.VMEM((CH, D), jnp.float32),
        pltpu.VMEM((CH, D), jnp.float32),
        pltpu.VMEM((CH,), jnp.float32),
        pltpu.VMEM((NPT,), jnp.float32),
        pltpu.VMEM_SHARED((NP_, D), jnp.float32),
        pltpu.VMEM_SHARED((NP_,), jnp.float32),
        pltpu.SemaphoreType.DMA,
        pltpu.SemaphoreType.DMA,
        pltpu.SemaphoreType.DMA,
        pltpu.SemaphoreType.DMA,
        pltpu.SemaphoreType.DMA,
    ],
)
def _sc_pass1(x_hbm, src_hbm, dst_hbm, z128_hbm, zdeg_hbm, ones_hbm,
              agg_out, deg_out, srca, srcb2, dstb, rowsa, rowsb, onesb, stgd,
              agg_sh, deg_sh, sga, sgb, sia, sib, sdeg):
    c = lax.axis_index("c")
    s = lax.axis_index("s")
    wid = c * NS + s
    pltpu.sync_copy(z128_hbm, rowsa)
    pltpu.sync_copy(zdeg_hbm, stgd)
    pltpu.sync_copy(ones_hbm, onesb)
    pltpu.sync_copy(dst_hbm.at[wid], dstb)
    for t in range(NZ):
        pltpu.sync_copy(rowsa, agg_sh.at[pl.ds(s * NPT + t * CH, CH)])
    pltpu.sync_copy(stgd, deg_sh.at[pl.ds(s * NPT, NPT)])
    pltpu.sync_copy(src_hbm.at[wid, 0], srca)
    pltpu.async_copy(x_hbm.at[srca], rowsa, sga)
    pltpu.async_copy(src_hbm.at[wid, 1], srcb2, sib)
    plsc.subcore_barrier()

    def body(t, carry):
        a = 2 * t
        b = a + 1

        @pl.when(b < NCH)
        def _():
            pltpu.make_async_copy(src_hbm.at[wid, b], srcb2, sib).wait()
            pltpu.async_copy(x_hbm.at[srcb2], rowsb, sgb)
        pltpu.make_async_copy(x_hbm.at[srca], rowsa, sga).wait()

        @pl.when(a + 2 < NCH)
        def _():
            pltpu.async_copy(src_hbm.at[wid, a + 2], srca, sia)
        pltpu.sync_copy(rowsa, agg_sh.at[dstb.at[a]], add=True)
        pltpu.async_copy(onesb, deg_sh.at[dstb.at[a]], sdeg, add=True)

        @pl.when(a + 2 < NCH)
        def _():
            pltpu.make_async_copy(src_hbm.at[wid, a + 2], srca, sia).wait()
            pltpu.async_copy(x_hbm.at[srca], rowsa, sga)

        @pl.when(b < NCH)
        def _():
            pltpu.make_async_copy(x_hbm.at[srcb2], rowsb, sgb).wait()

        @pl.when(b + 2 < NCH)
        def _():
            pltpu.async_copy(src_hbm.at[wid, b + 2], srcb2, sib)

        @pl.when(b < NCH)
        def _():
            pltpu.sync_copy(rowsb, agg_sh.at[dstb.at[b]], add=True)
            pltpu.async_copy(onesb, deg_sh.at[dstb.at[b]], sdeg, add=True)
        return carry

    lax.fori_loop(0, (NCH + 1) // 2, body, 0)

    def drain(j, carry):
        pltpu.make_async_copy(onesb, deg_sh.at[dstb.at[0]], sdeg).wait()
        return carry

    lax.fori_loop(0, NCH, drain, 0)
    plsc.subcore_barrier()
    for t in range(NZ):
        pltpu.sync_copy(agg_sh.at[pl.ds(s * NPT + t * CH, CH)], rowsa)
        pltpu.sync_copy(rowsa, agg_out.at[c, pl.ds(s * NPT + t * CH, CH)])
    pltpu.sync_copy(deg_sh.at[pl.ds(s * NPT, NPT)], stgd)
    pltpu.sync_copy(stgd, deg_out.at[c, pl.ds(s * NPT, NPT)])


@functools.partial(
    pl.kernel,
    mesh=_mesh,
    out_type=jax.ShapeDtypeStruct((NC, NP_), jnp.float32),
    compiler_params=pltpu.CompilerParams(needs_layout_passes=False),
    scratch_types=[
        pltpu.VMEM((N,), jnp.float32),
        pltpu.VMEM((NCH, CH), jnp.int32),
        pltpu.VMEM((NCH, CH), jnp.int32),
        [pltpu.VMEM((CH,), jnp.float32)] * 2,
        pltpu.VMEM((NPT,), jnp.float32),
        pltpu.VMEM_SHARED((NP_,), jnp.float32),
        [pltpu.SemaphoreType.DMA] * 2,
    ],
)
def _sc_pass2(hn_hbm, src_hbm, dst_hbm, zdeg_hbm,
              acc_out, hn_v, srcb, dstb, vals, stgd, acc_sh, ss):
    c = lax.axis_index("c")
    s = lax.axis_index("s")
    wid = c * NS + s
    pltpu.sync_copy(zdeg_hbm, stgd)
    pltpu.sync_copy(hn_hbm, hn_v)
    pltpu.sync_copy(src_hbm.at[wid], srcb)
    pltpu.sync_copy(dst_hbm.at[wid], dstb)
    pltpu.sync_copy(stgd, acc_sh.at[pl.ds(s * NPT, NPT)])
    plsc.subcore_barrier()

    def fill(q, ch):
        for kk in range(CH // 16):
            s16 = srcb[ch, pl.ds(kk * 16, 16)]
            g = plsc.load_gather(hn_v, [s16])
            vals[q][pl.ds(kk * 16, 16)] = g

    def body(t, carry):
        for q in range(2):
            ch = 2 * t + q

            @pl.when(ch < NCH)
            def _(q=q, ch=ch):
                @pl.when(ch >= 2)
                def _():
                    pltpu.make_async_copy(vals[q], acc_sh.at[dstb.at[ch - 2]], ss[q]).wait()
                fill(q, ch)
                pltpu.async_copy(vals[q], acc_sh.at[dstb.at[ch]], ss[q], add=True)
        return carry

    lax.fori_loop(0, (NCH + 1) // 2, body, 0)
    for q in range(2):
        last = ((NCH - 1 - q) // 2) * 2 + q
        pltpu.make_async_copy(vals[q], acc_sh.at[dstb.at[last]], ss[q]).wait()
    plsc.subcore_barrier()
    pltpu.sync_copy(acc_sh.at[pl.ds(s * NPT, NPT)], stgd)
    pltpu.sync_copy(stgd, acc_out.at[c, pl.ds(s * NPT, NPT)])


def _tc_a_body(x_ref, aggp_ref, degp_ref, ws_ref, wn_ref, b_ref,
               wscn_ref, wscs_ref, h_ref, hn_ref, hs_ref):
    x = x_ref[...]
    agg = aggp_ref[0] + aggp_ref[1]
    deg = jnp.maximum(degp_ref[0] + degp_ref[1], 1.0)
    m = agg / deg
    h = jnp.maximum(
        jnp.dot(x, ws_ref[...], preferred_element_type=jnp.float32)
        + jnp.dot(m, wn_ref[...], preferred_element_type=jnp.float32)
        + b_ref[...], 0.0)
    h_ref[...] = h
    hn_ref[...] = jnp.sum(h * wscn_ref[...], axis=1, keepdims=True)
    hs_ref[...] = jnp.sum(h * wscs_ref[...], axis=1, keepdims=True)


def _tc_a(x, aggp, degp3, ws, wn, b, wscn, wscs):
    blk = 1000
    grid = N // blk
    return pl.pallas_call(
        _tc_a_body,
        grid=(grid,),
        in_specs=[
            pl.BlockSpec((blk, D), lambda i: (i, 0)),
            pl.BlockSpec((NC, blk, D), lambda i: (0, i, 0)),
            pl.BlockSpec((NC, blk, 1), lambda i: (0, i, 0)),
            pl.BlockSpec((D, D), lambda i: (0, 0)),
            pl.BlockSpec((D, D), lambda i: (0, 0)),
            pl.BlockSpec((1, D), lambda i: (0, 0)),
            pl.BlockSpec((1, D), lambda i: (0, 0)),
            pl.BlockSpec((1, D), lambda i: (0, 0)),
        ],
        out_specs=[
            pl.BlockSpec((blk, D), lambda i: (i, 0)),
            pl.BlockSpec((blk, 1), lambda i: (i, 0)),
            pl.BlockSpec((blk, 1), lambda i: (i, 0)),
        ],
        out_shape=[
            jax.ShapeDtypeStruct((N, D), jnp.float32),
            jax.ShapeDtypeStruct((N, 1), jnp.float32),
            jax.ShapeDtypeStruct((N, 1), jnp.float32),
        ],
    )(x, aggp, degp3, ws, wn, b, wscn, wscs)


def _tc_b_body(h_ref, hs_ref, aggsp_ref, bsc_ref, gamma_ref, beta_ref,
               wfc_ref, bfc_ref, out_ref):
    hs = hs_ref[...]
    aggs = aggsp_ref[0] + aggsp_ref[1]
    sg = hs + aggs + bsc_ref[0, 0]
    iota = lax.broadcasted_iota(jnp.int32, (G, PER), 1)
    sel = jnp.zeros((G, PER), jnp.bool_)
    for _ in range(K):
        masked = jnp.where(sel, -jnp.inf, sg)
        mx = jnp.max(masked, axis=1, keepdims=True)
        pos = jnp.min(jnp.where(masked == mx, iota, PER), axis=1, keepdims=True)
        sel = jnp.logical_or(sel, iota == pos)
    w = jnp.where(sel, jnp.tanh(sg), 0.0)
    h = h_ref[...]
    S = jnp.zeros((G, D), jnp.float32)
    Q = jnp.zeros((G, D), jnp.float32)
    w2 = w * w
    for j in range(PER):
        hj = h[:, j, :]
        S = S + w[:, j:j + 1] * hj
        Q = Q + w2[:, j:j + 1] * (hj * hj)
    tot = jnp.float32(G * K)
    mu = jnp.sum(S, axis=0, keepdims=True) / tot
    q2 = jnp.sum(Q, axis=0, keepdims=True) / tot
    var = q2 - mu * mu
    rstd = lax.rsqrt(var + 1e-5)
    pooled = (S / jnp.float32(K) - mu) * rstd * gamma_ref[...] + beta_ref[...]
    logits = jnp.dot(pooled, wfc_ref[...], preferred_element_type=jnp.float32) + bfc_ref[...]
    lmx = jnp.max(logits, axis=1, keepdims=True)
    lse = jnp.log(jnp.sum(jnp.exp(logits - lmx), axis=1, keepdims=True)) + lmx
    out_ref[...] = logits - lse


def _tc_b(h3, hs2, aggsp, bsc, gamma, beta, wfc, bfc):
    return pl.pallas_call(
        _tc_b_body,
        out_shape=jax.ShapeDtypeStruct((G, OUT), jnp.float32),
    )(h3, hs2, aggsp, bsc, gamma, beta, wfc, bfc)


def kernel(x, edge_index, batch, W_enc_self, W_enc_nbr, b_enc,
           W_sc_self, W_sc_nbr, b_sc, gamma, beta, W_fc, b_fc):
    srcw = edge_index[0].astype(jnp.int32).reshape(NW, EPW)
    dstw = edge_index[1].astype(jnp.int32).reshape(NW, EPW)
    pad_src = ((jnp.arange(NW * PAD, dtype=jnp.int32) * 97) % N).reshape(NW, PAD)
    pad_dst = (N + (jnp.arange(NW * PAD, dtype=jnp.int32) % (NP_ - N))).reshape(NW, PAD)
    src3 = jnp.concatenate([srcw, pad_src], axis=1).reshape(NW, NCH, CH)
    dst3 = jnp.concatenate([dstw, pad_dst], axis=1).reshape(NW, NCH, CH)
    z128 = jnp.zeros((CH, D), jnp.float32)
    zdeg = jnp.zeros((NPT,), jnp.float32)
    ones = jnp.ones((CH,), jnp.float32)

    aggp, degp = _sc_pass1(x, src3, dst3, z128, zdeg, ones)
    h, hn, hs = _tc_a(x, aggp, degp.reshape(NC, NP_, 1)[:, :N, :],
                      W_enc_self, W_enc_nbr, b_enc.reshape(1, D),
                      W_sc_nbr.reshape(1, D), W_sc_self.reshape(1, D))
    aggsp = _sc_pass2(hn.reshape(N), src3, dst3, zdeg)
    out = _tc_b(h.reshape(G, PER, D), hs.reshape(G, PER),
                aggsp[:, :N].reshape(NC, G, PER), b_sc.reshape(1, 1),
                gamma.reshape(1, D), beta.reshape(1, D), W_fc,
                b_fc.reshape(1, OUT))
    return out
